# baseline (speedup 1.0000x reference)
import functools

import jax
import jax.numpy as jnp
from jax import lax
from jax.experimental import pallas as pl
from jax.experimental.pallas import tpu as pltpu
from jax.experimental.pallas import tpu_sc as plsc

N, D, E, H, TOPK = 2048, 768, 8, 1536, 2
NK = N * TOPK
BM = 512
NW = 32
TPW = N // NW
D2 = D // 2


def _route_body(x_ref, wg_ref, bg_ref,
                w0_ref, w1_ref, d0_ref, d1_ref, offs_ref):
    logits = jnp.dot(x_ref[...], wg_ref[...],
                     preferred_element_type=jnp.float32) + bg_ref[...]
    m = jnp.max(logits, axis=-1, keepdims=True)
    p = jnp.exp(logits - m)
    p = p / jnp.sum(p, axis=-1, keepdims=True)
    eidx = jax.lax.broadcasted_iota(jnp.int32, p.shape, 1)
    big = jnp.int32(E)
    p1 = jnp.max(p, axis=-1, keepdims=True)
    i1 = jnp.min(jnp.where(p == p1, eidx, big), axis=-1, keepdims=True)
    mask1 = eidx == i1
    pm = jnp.where(mask1, -jnp.inf, p)
    p2 = jnp.max(pm, axis=-1, keepdims=True)
    i2 = jnp.min(jnp.where(pm == p2, eidx, big), axis=-1, keepdims=True)
    mask2 = eidx == i2
    denom = p1 + p2
    w0_ref[...] = p1 / denom
    w1_ref[...] = p2 / denom

    oh1 = mask1.astype(jnp.float32)
    oh2 = mask2.astype(jnp.float32)

    def excl_cumsum(a):
        s = a
        for sh in (1, 2, 4, 8, 16, 32, 64, 128, 256, 512, 1024):
            s = s + jnp.concatenate(
                [jnp.zeros((sh, E), jnp.float32), s[:N - sh]], axis=0)
        return s - a

    c1 = excl_cumsum(oh1)
    tot1 = jnp.sum(oh1, axis=0, keepdims=True)
    c2 = excl_cumsum(oh2) + tot1
    counts = tot1 + jnp.sum(oh2, axis=0, keepdims=True)

    r8 = jax.lax.broadcasted_iota(jnp.int32, (E, E), 0)
    c8 = jax.lax.broadcasted_iota(jnp.int32, (E, E), 1)
    counts_col = jnp.sum(jnp.where(r8 == c8, counts, 0.0),
                         axis=1, keepdims=True)
    offs = jnp.sum(jnp.where(r8 < c8, counts_col, 0.0),
                   axis=0, keepdims=True)
    offs_ref[...] = offs.astype(jnp.int32)

    d0 = jnp.sum(jnp.where(mask1, offs + c1, 0.0), axis=1, keepdims=True)
    d1 = jnp.sum(jnp.where(mask2, offs + c2, 0.0), axis=1, keepdims=True)
    d0_ref[...] = d0.astype(jnp.int32)
    d1_ref[...] = d1.astype(jnp.int32)


def _route(x, Wg, bg):
    return pl.pallas_call(
        _route_body,
        grid=(1,),
        in_specs=[
            pl.BlockSpec((N, D), lambda i: (0, 0)),
            pl.BlockSpec((D, E), lambda i: (0, 0)),
            pl.BlockSpec((E,), lambda i: (0,)),
        ],
        out_specs=[
            pl.BlockSpec((N, 1), lambda i: (0, 0)),
            pl.BlockSpec((N, 1), lambda i: (0, 0)),
            pl.BlockSpec((N, 1), lambda i: (0, 0)),
            pl.BlockSpec((N, 1), lambda i: (0, 0)),
            pl.BlockSpec((1, E), lambda i: (0, 0)),
        ],
        out_shape=[
            jax.ShapeDtypeStruct((N, 1), jnp.float32),
            jax.ShapeDtypeStruct((N, 1), jnp.float32),
            jax.ShapeDtypeStruct((N, 1), jnp.int32),
            jax.ShapeDtypeStruct((N, 1), jnp.int32),
            jax.ShapeDtypeStruct((1, E), jnp.int32),
        ],
    )(x, Wg, bg)


@functools.lru_cache(maxsize=1)
def _sc_kernels():
    mesh = plsc.VectorSubcoreMesh(core_axis_name="c", subcore_axis_name="s")

    @functools.partial(
        pl.kernel, mesh=mesh,
        out_type=jax.ShapeDtypeStruct((NK, D2), jnp.int32),
        scratch_types=[
            pltpu.VMEM((TPW,), jnp.int32),
            pltpu.VMEM((TPW,), jnp.int32),
            pltpu.VMEM((TPW, D2), jnp.int32),
            pltpu.SemaphoreType.DMA,
        ],
    )
    def sc_scatter(x_hbm, d0_hbm, d1_hbm, xs_hbm, i0_v, i1_v, rows_v, sem):
        wid = lax.axis_index("s") * 2 + lax.axis_index("c")
        base = wid * TPW
        pltpu.sync_copy(d0_hbm.at[pl.ds(base, TPW)], i0_v)
        pltpu.sync_copy(d1_hbm.at[pl.ds(base, TPW)], i1_v)
        pltpu.sync_copy(x_hbm.at[pl.ds(base, TPW)], rows_v)
        cp0 = pltpu.async_copy(rows_v, xs_hbm.at[i0_v], sem)
        cp1 = pltpu.async_copy(rows_v, xs_hbm.at[i1_v], sem)
        cp0.wait()
        cp1.wait()

    @functools.partial(
        pl.kernel, mesh=mesh,
        out_type=[
            jax.ShapeDtypeStruct((N, D2), jnp.int32),
            jax.ShapeDtypeStruct((N, D2), jnp.int32),
        ],
        scratch_types=[
            pltpu.VMEM((TPW,), jnp.int32),
            pltpu.VMEM((TPW,), jnp.int32),
            pltpu.VMEM((TPW, D2), jnp.int32),
            pltpu.VMEM((TPW, D2), jnp.int32),
            pltpu.SemaphoreType.DMA,
            pltpu.SemaphoreType.DMA,
        ],
    )
    def sc_gather(ys_hbm, d0_hbm, d1_hbm, z0_hbm, z1_hbm,
                  i0_v, i1_v, r0_v, r1_v, sem0, sem1):
        wid = lax.axis_index("s") * 2 + lax.axis_index("c")
        base = wid * TPW
        pltpu.sync_copy(d0_hbm.at[pl.ds(base, TPW)], i0_v)
        pltpu.sync_copy(d1_hbm.at[pl.ds(base, TPW)], i1_v)
        cp0 = pltpu.async_copy(ys_hbm.at[i0_v], r0_v, sem0)
        cp1 = pltpu.async_copy(ys_hbm.at[i1_v], r1_v, sem1)
        cp0.wait()
        pltpu.sync_copy(r0_v, z0_hbm.at[pl.ds(base, TPW)])
        cp1.wait()
        pltpu.sync_copy(r1_v, z1_hbm.at[pl.ds(base, TPW)])

    return sc_scatter, sc_gather


def _sc_scatter(x, d0f, d1f):
    return _sc_kernels()[0](x, d0f, d1f)


def _sc_gather(ys, d0f, d1f):
    return _sc_kernels()[1](ys, d0f, d1f)


def _gmm_body(offs_ref, xs_ref, w1_ref, b1_ref, w2_ref, b2_ref,
              ys_ref, h_ref):
    b = pl.program_id(0)
    row0 = b * BM
    riota = jax.lax.broadcasted_iota(jnp.int32, (BM, 1), 0) + row0
    xb = xs_ref[...]
    for e in range(E):
        start = offs_ref[0, e]
        end = offs_ref[0, e + 1] if e < E - 1 else jnp.int32(NK)

        @pl.when((end > row0) & (start < row0 + BM))
        def _mm1(e=e, start=start, end=end):
            contrib = jnp.dot(xb, w1_ref[e],
                              preferred_element_type=jnp.float32)
            contrib = contrib + b1_ref[e:e + 1, :]
            msk = (riota >= start) & (riota < end)
            h_ref[...] = jnp.where(msk, contrib, h_ref[...])

    a = (0.5 * h_ref[...] *
         (1.0 + jax.lax.erf(h_ref[...] * 0.7071067811865476))
         ).astype(jnp.bfloat16)
    for e in range(E):
        start = offs_ref[0, e]
        end = offs_ref[0, e + 1] if e < E - 1 else jnp.int32(NK)

        @pl.when((end > row0) & (start < row0 + BM))
        def _mm2(e=e, start=start, end=end):
            contrib = jnp.dot(a, w2_ref[e],
                              preferred_element_type=jnp.float32)
            contrib = (contrib + b2_ref[e:e + 1, :]).astype(jnp.bfloat16)
            msk = (riota >= start) & (riota < end)
            ys_ref[...] = jnp.where(msk, contrib, ys_ref[...])


def _gmm(offs, xs, w1b, b1, w2b, b2):
    return pl.pallas_call(
        _gmm_body,
        grid=(NK // BM,),
        in_specs=[
            pl.BlockSpec(memory_space=pltpu.SMEM),
            pl.BlockSpec((BM, D), lambda b: (b, 0)),
            pl.BlockSpec((E, D, H), lambda b: (0, 0, 0)),
            pl.BlockSpec((E, H), lambda b: (0, 0)),
            pl.BlockSpec((E, H, D), lambda b: (0, 0, 0)),
            pl.BlockSpec((E, D), lambda b: (0, 0)),
        ],
        out_specs=pl.BlockSpec((BM, D), lambda b: (b, 0)),
        out_shape=jax.ShapeDtypeStruct((NK, D), jnp.bfloat16),
        scratch_shapes=[
            pltpu.VMEM((BM, H), jnp.float32),
        ],
        compiler_params=pltpu.CompilerParams(
            dimension_semantics=("arbitrary",),
        ),
    )(offs, xs, w1b, b1, w2b, b2)


def _combine_body(x_ref, w0_ref, w1_ref, z0_ref, z1_ref, out_ref):
    out_ref[...] = (x_ref[...]
                    + w0_ref[...] * z0_ref[...].astype(jnp.float32)
                    + w1_ref[...] * z1_ref[...].astype(jnp.float32))


def _combine(x, w0, w1, z0, z1):
    return pl.pallas_call(
        _combine_body,
        grid=(2,),
        in_specs=[
            pl.BlockSpec((N // 2, D), lambda i: (i, 0)),
            pl.BlockSpec((N // 2, 1), lambda i: (i, 0)),
            pl.BlockSpec((N // 2, 1), lambda i: (i, 0)),
            pl.BlockSpec((N // 2, D), lambda i: (i, 0)),
            pl.BlockSpec((N // 2, D), lambda i: (i, 0)),
        ],
        out_specs=pl.BlockSpec((N // 2, D), lambda i: (i, 0)),
        out_shape=jax.ShapeDtypeStruct((N, D), jnp.float32),
    )(x, w0, w1, z0, z1)


def _pack(a_bf16):
    r = a_bf16.shape[0]
    return jax.lax.bitcast_convert_type(
        a_bf16.reshape(r, D2, 2), jnp.int32)


def _unpack(a_i32):
    r = a_i32.shape[0]
    return jax.lax.bitcast_convert_type(a_i32, jnp.bfloat16).reshape(r, D)


@jax.jit
def kernel(x, Wg, bg, W1, b1, W2, b2):
    w0, w1, d0, d1, offs = _route(x, Wg, bg)
    d0f = d0.reshape(N)
    d1f = d1.reshape(N)
    xs = _unpack(_sc_scatter(_pack(x.astype(jnp.bfloat16)), d0f, d1f))
    w1b = W1.astype(jnp.bfloat16)
    w2b = W2.astype(jnp.bfloat16)
    ys = _gmm(offs, xs, w1b, b1, w2b, b2)
    z0, z1 = _sc_gather(_pack(ys), d0f, d1f)
    return _combine(x, w0, w1, _unpack(z0), _unpack(z1))


# speedup vs baseline: 3.9637x; 3.9637x over previous
import jax
import jax.numpy as jnp
from jax.experimental import pallas as pl
from jax.experimental.pallas import tpu as pltpu

N, D, E, H, TOPK = 2048, 768, 8, 1536, 2
BLK_N = 1024


def _moe_body(x_ref, wg_ref, bg_ref, w1_ref, b1_ref, w2_ref, b2_ref,
              out_ref, gate_ref, acc_ref):
    e = pl.program_id(1)

    @pl.when(e == 0)
    def _gate():
        xb = x_ref[...]
        logits = jnp.dot(xb, wg_ref[...],
                         preferred_element_type=jnp.float32) + bg_ref[...]
        m = jnp.max(logits, axis=-1, keepdims=True)
        p = jnp.exp(logits - m)
        p = p / jnp.sum(p, axis=-1, keepdims=True)
        eidx = jax.lax.broadcasted_iota(jnp.int32, p.shape, 1)
        big = jnp.int32(E)
        p1 = jnp.max(p, axis=-1, keepdims=True)
        i1 = jnp.min(jnp.where(p == p1, eidx, big), axis=-1, keepdims=True)
        mask1 = eidx == i1
        pm = jnp.where(mask1, -jnp.inf, p)
        p2 = jnp.max(pm, axis=-1, keepdims=True)
        i2 = jnp.min(jnp.where(pm == p2, eidx, big), axis=-1, keepdims=True)
        mask2 = eidx == i2
        denom = p1 + p2
        gate_ref[...] = jnp.where(mask1 | mask2, p / denom, 0.0)
        acc_ref[...] = xb

    xb16 = x_ref[...].astype(jnp.bfloat16)
    w1e = w1_ref[0].astype(jnp.bfloat16)
    w2e = w2_ref[0].astype(jnp.bfloat16)
    b1e = b1_ref[pl.ds(e, 1), :]
    b2e = b2_ref[pl.ds(e, 1), :]
    h = jnp.dot(xb16, w1e, preferred_element_type=jnp.float32) + b1e
    a = (0.5 * h * (1.0 + jax.lax.erf(h * 0.7071067811865476))
         ).astype(jnp.bfloat16)
    y = jnp.dot(a, w2e, preferred_element_type=jnp.float32) + b2e
    gate = gate_ref[...]
    col = jax.lax.broadcasted_iota(jnp.int32, gate.shape, 1)
    w_e = jnp.sum(jnp.where(col == e, gate, 0.0), axis=1, keepdims=True)
    acc_ref[...] += w_e * y

    @pl.when(e == E - 1)
    def _write():
        out_ref[...] = acc_ref[...]


@jax.jit
def kernel(x, Wg, bg, W1, b1, W2, b2):
    grid = (N // BLK_N, E)
    out = pl.pallas_call(
        _moe_body,
        grid=grid,
        in_specs=[
            pl.BlockSpec((BLK_N, D), lambda n, e: (n, 0)),
            pl.BlockSpec((D, E), lambda n, e: (0, 0)),
            pl.BlockSpec((E,), lambda n, e: (0,)),
            pl.BlockSpec((1, D, H), lambda n, e: (e, 0, 0)),
            pl.BlockSpec((E, H), lambda n, e: (0, 0)),
            pl.BlockSpec((1, H, D), lambda n, e: (e, 0, 0)),
            pl.BlockSpec((E, D), lambda n, e: (0, 0)),
        ],
        out_specs=pl.BlockSpec((BLK_N, D), lambda n, e: (n, 0)),
        out_shape=jax.ShapeDtypeStruct((N, D), jnp.float32),
        scratch_shapes=[
            pltpu.VMEM((BLK_N, E), jnp.float32),
            pltpu.VMEM((BLK_N, D), jnp.float32),
        ],
        compiler_params=pltpu.CompilerParams(
            dimension_semantics=("arbitrary", "arbitrary"),
        ),
    )(x, Wg, bg, W1, b1, W2, b2)
    return out
